# baseline (speedup 1.0000x reference)
import functools

import jax
import jax.numpy as jnp
from jax import lax
from jax.experimental import pallas as pl
from jax.experimental.pallas import tpu as pltpu
from jax.experimental.pallas import tpu_sc as plsc

NUM_USERS_ROWS = 100
NUM_ITEMS_ROWS = 500
EMB_DIM = 16


def _copy_tables(user_hbm, item_hbm, user_out, item_out):
    cid = lax.axis_index("c")

    @pl.when(cid == 0)
    def _():
        pltpu.sync_copy(user_hbm, user_out)

    @pl.when(cid == 1)
    def _():
        pltpu.sync_copy(item_hbm, item_out)


@jax.jit
def kernel(graph_data, user_emb, item_emb):
    del graph_data
    mesh = plsc.ScalarSubcoreMesh(axis_name="c", num_cores=2)
    copy = functools.partial(
        pl.kernel,
        out_type=(
            jax.ShapeDtypeStruct((NUM_USERS_ROWS, EMB_DIM), jnp.float32),
            jax.ShapeDtypeStruct((NUM_ITEMS_ROWS, EMB_DIM), jnp.float32),
        ),
        mesh=mesh,
    )(_copy_tables)
    return copy(user_emb, item_emb)


# speedup vs baseline: 1.0734x; 1.0734x over previous
import functools

import jax
import jax.numpy as jnp
from jax import lax
from jax.experimental import pallas as pl
from jax.experimental.pallas import tpu as pltpu
from jax.experimental.pallas import tpu_sc as plsc

NUM_USERS_ROWS = 100
NUM_ITEMS_ROWS = 500
EMB_DIM = 16


def _copy_tables(user_hbm, item_hbm, user_out, item_out, sem_u, sem_i):
    cu = pltpu.make_async_copy(user_hbm, user_out, sem_u)
    ci = pltpu.make_async_copy(item_hbm, item_out, sem_i)
    cu.start()
    ci.start()
    cu.wait()
    ci.wait()


@jax.jit
def kernel(graph_data, user_emb, item_emb):
    del graph_data
    mesh = plsc.ScalarSubcoreMesh(axis_name="c", num_cores=1)
    copy = functools.partial(
        pl.kernel,
        out_type=(
            jax.ShapeDtypeStruct((NUM_USERS_ROWS, EMB_DIM), jnp.float32),
            jax.ShapeDtypeStruct((NUM_ITEMS_ROWS, EMB_DIM), jnp.float32),
        ),
        scratch_types=[pltpu.SemaphoreType.DMA, pltpu.SemaphoreType.DMA],
        mesh=mesh,
    )(_copy_tables)
    return copy(user_emb, item_emb)
